# Initial kernel scaffold; baseline (speedup 1.0000x reference)
import jax
import jax.numpy as jnp
from jax.experimental import pallas as pl


def kernel(x, pos_table, tt_table, gamma, beta):
    raise NotImplementedError("write your pallas kernel here")


# speedup vs baseline: 4.8140x; 4.8140x over previous
import functools

import jax
import jax.numpy as jnp
from jax.experimental import pallas as pl
from jax.experimental.pallas import tpu as pltpu


def _ln_body(x_ref, pos_ref, tt_ref, gamma_ref, beta_ref, o_ref):
    x = x_ref[...]
    add = pos_ref[...] + tt_ref[...]
    emb = x + add[:, None, :]
    mean = jnp.mean(emb, axis=-1, keepdims=True)
    c = emb - mean
    var = jnp.mean(c * c, axis=-1, keepdims=True)
    rstd = jax.lax.rsqrt(var + 1e-12)
    g = gamma_ref[...][None, :, :]
    b = beta_ref[...][None, :, :]
    o_ref[...] = c * rstd * g + b


@functools.partial(jax.jit, static_argnames=())
def kernel(x, pos_table, tt_table, gamma, beta):
    S, B, D = x.shape
    Sb = 256
    grid = (S // Sb,)
    tt_row = tt_table[0:1]
    gamma2 = gamma.reshape(1, D)
    beta2 = beta.reshape(1, D)
    out = pl.pallas_call(
        _ln_body,
        grid=grid,
        in_specs=[
            pl.BlockSpec((Sb, B, D), lambda i: (i, 0, 0)),
            pl.BlockSpec((Sb, D), lambda i: (i, 0)),
            pl.BlockSpec((1, D), lambda i: (0, 0)),
            pl.BlockSpec((1, D), lambda i: (0, 0)),
            pl.BlockSpec((1, D), lambda i: (0, 0)),
        ],
        out_specs=pl.BlockSpec((Sb, B, D), lambda i: (i, 0, 0)),
        out_shape=jax.ShapeDtypeStruct((S, B, D), x.dtype),
        compiler_params=pltpu.CompilerParams(
            dimension_semantics=("arbitrary",),
        ),
    )(x, pos_table, tt_row, gamma2, beta2)
    return out
